# Initial kernel scaffold; baseline (speedup 1.0000x reference)
import jax
import jax.numpy as jnp
from jax.experimental import pallas as pl


def kernel(x, lin1_w, lin1_b, conv_w, lin2_w, lin2_b, edge_index, batch):
    raise NotImplementedError("write your pallas kernel here")


# speedup vs baseline: 5.1659x; 5.1659x over previous
import functools

import jax
import jax.numpy as jnp
import numpy as np
from jax import lax
from jax.experimental import pallas as pl
from jax.experimental.pallas import tpu as pltpu
from jax.experimental.pallas import tpu_sc as plsc

N = 10000
E = 320000
H = 128
C = 16
L = 8
G = 64
ALPHA = 0.1
THETA = 0.5

NC = 2
NS = 16
NW = NC * NS
BLK = 128
BPT = 80
EPAD = NW * BPT * BLK
NPAD = 10240
SPAN = NPAD // NS
RB = 1000
NRB = N // RB


def _zero_vec16(ref):
    n = ref.shape[0]
    z = jnp.zeros((16,), jnp.float32)

    @pl.loop(0, n // 16)
    def _(j):
        ref[pl.ds(j * 16, 16)] = z


_SC_MESH = dict(core_axis_name="c", subcore_axis_name="s")


@functools.partial(
    pl.kernel,
    out_type=jax.ShapeDtypeStruct((NC, NPAD), jnp.float32),
    mesh=plsc.VectorSubcoreMesh(**_SC_MESH),
    scratch_types=[
        pltpu.VMEM((BPT, BLK), jnp.int32),
        pltpu.VMEM((BLK,), jnp.float32),
        pltpu.VMEM((SPAN,), jnp.float32),
        pltpu.VMEM_SHARED((NPAD,), jnp.float32),
    ],
)
def _sc_degree(dst_hbm, out_hbm, dst_v, ones_v, stage_v, deg_sh):
    c = lax.axis_index("c")
    s = lax.axis_index("s")
    wid = s * NC + c
    base = s * SPAN

    pltpu.sync_copy(dst_hbm.at[wid], dst_v)

    @pl.loop(0, BLK // 16)
    def _(j):
        ones_v[pl.ds(j * 16, 16)] = jnp.full((16,), 1.0, jnp.float32)

    _zero_vec16(stage_v)
    pltpu.sync_copy(stage_v, deg_sh.at[pl.ds(base, SPAN)])
    plsc.subcore_barrier()

    @pl.loop(0, BPT)
    def _(j):
        pltpu.sync_copy(ones_v, deg_sh.at[dst_v.at[j]], add=True)

    plsc.subcore_barrier()
    pltpu.sync_copy(deg_sh.at[pl.ds(base, SPAN)], stage_v)
    pltpu.sync_copy(stage_v, out_hbm.at[c, pl.ds(base, SPAN)])


@functools.partial(
    pl.kernel,
    out_type=jax.ShapeDtypeStruct((NC, NPAD, H), jnp.float32),
    mesh=plsc.VectorSubcoreMesh(**_SC_MESH),
    scratch_types=[
        pltpu.VMEM((BPT, BLK), jnp.int32),
        pltpu.VMEM((BPT, BLK), jnp.int32),
        pltpu.VMEM((BLK, H), jnp.float32),
        pltpu.VMEM_SHARED((NPAD, H), jnp.float32),
        pltpu.SemaphoreType.DMA,
    ],
)
def _sc_scatter(hs_hbm, src_hbm, dst_hbm, out_hbm, src_v, dst_v, rows_v,
                agg_sh, sem):
    c = lax.axis_index("c")
    s = lax.axis_index("s")
    wid = s * NC + c
    base = s * SPAN

    pltpu.sync_copy(src_hbm.at[wid], src_v)
    pltpu.sync_copy(dst_hbm.at[wid], dst_v)

    @pl.loop(0, BLK)
    def _(i):
        _zero_vec16(rows_v.at[i])

    @pl.loop(0, SPAN // BLK)
    def _(k):
        pltpu.sync_copy(rows_v, agg_sh.at[pl.ds(base + k * BLK, BLK)])

    plsc.subcore_barrier()

    @pl.loop(0, BPT)
    def _(j):
        pltpu.async_copy(hs_hbm.at[src_v.at[j]], rows_v, sem).wait()
        pltpu.sync_copy(rows_v, agg_sh.at[dst_v.at[j]], add=True)

    plsc.subcore_barrier()

    @pl.loop(0, SPAN // BLK)
    def _(k):
        off = base + k * BLK
        pltpu.sync_copy(agg_sh.at[pl.ds(off, BLK)], rows_v)
        pltpu.sync_copy(rows_v, out_hbm.at[c, pl.ds(off, BLK)])


def _tc_front(x, w1, b1, d0, d1):
    def body(x_ref, w_ref, b_ref, d0_ref, d1_ref, h_ref, hs_ref, dinv_ref):
        deg = d0_ref[...] + d1_ref[...] + 1.0
        dinv = lax.rsqrt(deg)
        h = jnp.dot(x_ref[...], w_ref[...], preferred_element_type=jnp.float32)
        h = jnp.maximum(h + b_ref[...], 0.0)
        h_ref[...] = h
        hs_ref[...] = h * dinv
        dinv_ref[...] = dinv

    return pl.pallas_call(
        body,
        grid=(NRB,),
        in_specs=[
            pl.BlockSpec((RB, H), lambda i: (i, 0)),
            pl.BlockSpec((H, H), lambda i: (0, 0)),
            pl.BlockSpec((1, H), lambda i: (0, 0)),
            pl.BlockSpec((RB, 1), lambda i: (i, 0)),
            pl.BlockSpec((RB, 1), lambda i: (i, 0)),
        ],
        out_specs=[
            pl.BlockSpec((RB, H), lambda i: (i, 0)),
            pl.BlockSpec((RB, H), lambda i: (i, 0)),
            pl.BlockSpec((RB, 1), lambda i: (i, 0)),
        ],
        out_shape=[
            jax.ShapeDtypeStruct((N, H), jnp.float32),
            jax.ShapeDtypeStruct((N, H), jnp.float32),
            jax.ShapeDtypeStruct((N, 1), jnp.float32),
        ],
    )(x, w1, b1, d0, d1)


def _tc_layer(a0, a1, hs, x0, dinv, w, beta):
    def body(a0_ref, a1_ref, hs_ref, x0_ref, dinv_ref, w_ref, b_ref,
             h_ref, hsn_ref):
        beta_v = b_ref[0, 0]
        agg = (a0_ref[...] + a1_ref[...] + hs_ref[...]) * dinv_ref[...]
        hh = (1.0 - ALPHA) * agg + ALPHA * x0_ref[...]
        mm = jnp.dot(hh, w_ref[...], preferred_element_type=jnp.float32)
        h = jnp.maximum((1.0 - beta_v) * hh + beta_v * mm, 0.0)
        h_ref[...] = h
        hsn_ref[...] = h * dinv_ref[...]

    return pl.pallas_call(
        body,
        grid=(NRB,),
        in_specs=[
            pl.BlockSpec((RB, H), lambda i: (i, 0)),
            pl.BlockSpec((RB, H), lambda i: (i, 0)),
            pl.BlockSpec((RB, H), lambda i: (i, 0)),
            pl.BlockSpec((RB, H), lambda i: (i, 0)),
            pl.BlockSpec((RB, 1), lambda i: (i, 0)),
            pl.BlockSpec((H, H), lambda i: (0, 0)),
            pl.BlockSpec((1, 1), lambda i: (0, 0)),
        ],
        out_specs=[
            pl.BlockSpec((RB, H), lambda i: (i, 0)),
            pl.BlockSpec((RB, H), lambda i: (i, 0)),
        ],
        out_shape=[
            jax.ShapeDtypeStruct((N, H), jnp.float32),
            jax.ShapeDtypeStruct((N, H), jnp.float32),
        ],
    )(a0, a1, hs, x0, dinv, w, beta)


def _tc_pool_head(batch3, h, w2, b2):
    def body(b3_ref, h_ref, w2_ref, b2_ref, o_ref, sums, counts):
        i = pl.program_id(0)

        @pl.when(i == 0)
        def _():
            sums[...] = jnp.zeros_like(sums)
            counts[...] = jnp.zeros_like(counts)

        bvec = b3_ref[...].reshape(1, RB)
        ids = lax.broadcasted_iota(jnp.int32, (G, RB), 0)
        mask = jnp.where(ids == bvec, 1.0, 0.0)
        sums[...] += jnp.dot(mask, h_ref[...],
                             preferred_element_type=jnp.float32)
        counts[...] += jnp.sum(mask, axis=1, keepdims=True)

        @pl.when(i == NRB - 1)
        def _():
            pooled = sums[...] / jnp.maximum(counts[...], 1.0)
            logits = jnp.dot(pooled, w2_ref[...],
                             preferred_element_type=jnp.float32) + b2_ref[...]
            m = jnp.max(logits, axis=-1, keepdims=True)
            e = logits - m
            o_ref[...] = e - jnp.log(jnp.sum(jnp.exp(e), axis=-1,
                                             keepdims=True))

    return pl.pallas_call(
        body,
        grid=(NRB,),
        in_specs=[
            pl.BlockSpec((1, 1, RB), lambda i: (i, 0, 0)),
            pl.BlockSpec((RB, H), lambda i: (i, 0)),
            pl.BlockSpec((H, C), lambda i: (0, 0)),
            pl.BlockSpec((1, C), lambda i: (0, 0)),
        ],
        out_specs=pl.BlockSpec((G, C), lambda i: (0, 0)),
        out_shape=jax.ShapeDtypeStruct((G, C), jnp.float32),
        scratch_shapes=[
            pltpu.VMEM((G, H), jnp.float32),
            pltpu.VMEM((G, 1), jnp.float32),
        ],
    )(batch3, h, w2, b2)


def kernel(x, lin1_w, lin1_b, conv_w, lin2_w, lin2_b, edge_index, batch):
    pad = EPAD - E
    src_p = jnp.concatenate(
        [edge_index[0].astype(jnp.int32),
         jnp.zeros((pad,), jnp.int32)]).reshape(NW, BPT, BLK)
    dst_p = jnp.concatenate(
        [edge_index[1].astype(jnp.int32),
         jnp.full((pad,), N, jnp.int32)]).reshape(NW, BPT, BLK)

    degraw = _sc_degree(dst_p)
    d0 = degraw[0, :N, None]
    d1 = degraw[1, :N, None]

    h, hs, dinv = _tc_front(x, lin1_w, lin1_b.reshape(1, H), d0, d1)
    x0 = h
    for l in range(L):
        beta = float(np.log(THETA / (l + 1) + 1.0))
        araw = _sc_scatter(hs, src_p, dst_p)
        h, hs = _tc_layer(araw[0, :N], araw[1, :N], hs, x0, dinv,
                          conv_w[l], jnp.full((1, 1), beta, jnp.float32))

    batch3 = batch.astype(jnp.int32).reshape(NRB, 1, RB)
    return _tc_pool_head(batch3, h, lin2_w, lin2_b.reshape(1, C))
